# baseline (speedup 1.0000x reference)
import functools

import jax
import jax.numpy as jnp
from jax import lax
from jax.experimental import pallas as pl
from jax.experimental.pallas import tpu as pltpu
from jax.experimental.pallas import tpu_sc as plsc

VOCAB = 1000000
D = 64
B = 4096
L = 200

NC = 2
NS = 16
LANES = 16
NW = NC * NS
NB = B // NW
C0 = 104
C1 = L - C0
ND = D // LANES


def _bag_kernel(table_hbm, batch_hbm, len_hbm, inv_hbm, out_hbm,
                idx0_v, idx1_v, rows0_v, rows1_v, len_v, inv_v, pooled_v,
                sem0, sem1):
    wid = lax.axis_index("s") * NC + lax.axis_index("c")
    base = wid * NB

    pltpu.sync_copy(len_hbm.at[pl.ds(base, NB)], len_v.at[pl.ds(0, NB)])
    pltpu.sync_copy(inv_hbm.at[pl.ds(base, NB)], inv_v.at[pl.ds(0, NB)])

    def bag_body(i, carry):
        row_off = (base + i) * L
        pltpu.sync_copy(batch_hbm.at[pl.ds(row_off, C0)], idx0_v)
        pltpu.sync_copy(batch_hbm.at[pl.ds(row_off + C0, C1)], idx1_v)
        cp0 = pltpu.async_copy(table_hbm.at[idx0_v], rows0_v, sem0)
        cp1 = pltpu.async_copy(table_hbm.at[idx1_v], rows1_v, sem1)
        cp0.wait()
        cp1.wait()

        length = len_v[pl.ds(i, LANES)][0]
        n0 = jnp.minimum(length, C0)
        n1 = jnp.maximum(length - C0, 0)

        def acc0(j, acc):
            return tuple(acc[d] + rows0_v[j, pl.ds(d * LANES, LANES)]
                         for d in range(ND))

        def acc1(j, acc):
            return tuple(acc[d] + rows1_v[j, pl.ds(d * LANES, LANES)]
                         for d in range(ND))

        zeros = tuple(jnp.zeros((LANES,), jnp.float32) for _ in range(ND))
        acc = lax.fori_loop(0, n0, acc0, zeros)
        acc = lax.fori_loop(0, n1, acc1, acc)

        inv = inv_v[pl.ds(i, LANES)][0]
        for d in range(ND):
            pooled_v[i, pl.ds(d * LANES, LANES)] = acc[d] * inv
        return carry

    lax.fori_loop(0, NB, bag_body, 0)
    pltpu.sync_copy(pooled_v, out_hbm.at[pl.ds(base, NB), :])


def _embedding_bag(table, batch, lengths, inv):
    mesh = plsc.VectorSubcoreMesh(core_axis_name="c", subcore_axis_name="s")
    f = functools.partial(
        pl.kernel,
        mesh=mesh,
        out_type=jax.ShapeDtypeStruct((B, D), jnp.float32),
        compiler_params=pltpu.CompilerParams(use_tc_tiling_on_sc=False),
        scratch_types=[
            pltpu.VMEM((C0,), jnp.int32),
            pltpu.VMEM((C1,), jnp.int32),
            pltpu.VMEM((C0, D), jnp.float32),
            pltpu.VMEM((C1, D), jnp.float32),
            pltpu.VMEM((NB + LANES,), jnp.int32),
            pltpu.VMEM((NB + LANES,), jnp.float32),
            pltpu.VMEM((NB, D), jnp.float32),
            pltpu.SemaphoreType.DMA,
            pltpu.SemaphoreType.DMA,
        ],
    )(_bag_kernel)
    return f(table, batch, lengths, inv)


def _mlp_kernel(x_ref, w1_ref, b1_ref, w2_ref, b2_ref, o_ref):
    x = x_ref[...]
    h = lax.dot_general(x, w1_ref[...], (((1,), (1,)), ((), ())),
                        preferred_element_type=jnp.float32)
    h = jnp.maximum(h + b1_ref[...], 0.0)
    o = lax.dot_general(h, w2_ref[...], (((1,), (1,)), ((), ())),
                        preferred_element_type=jnp.float32)
    o_ref[...] = o + b2_ref[...]


def _mlp(pooled, W1, b1, W2, b2):
    return pl.pallas_call(
        _mlp_kernel,
        out_shape=jax.ShapeDtypeStruct((B, 2), jnp.float32),
    )(pooled, W1, b1.reshape(1, -1), W2, b2.reshape(1, -1))


@jax.jit
def kernel(table, batch, lengths, W1, b1, W2, b2):
    inv = 1.0 / jnp.clip(lengths, 1, None).astype(jnp.float32)
    pooled = _embedding_bag(table, batch.reshape(-1), lengths, inv)
    return _mlp(pooled, W1, b1, W2, b2)


# speedup vs baseline: 1.2687x; 1.2687x over previous
import functools

import jax
import jax.numpy as jnp
from jax import lax
from jax.experimental import pallas as pl
from jax.experimental.pallas import tpu as pltpu
from jax.experimental.pallas import tpu_sc as plsc

VOCAB = 1000000
D = 64
B = 4096
L = 200

NC = 2
NS = 16
LANES = 16
NW = NC * NS
NB = B // NW
C0 = 104
C1 = L - C0
ND = D // LANES


def _bag_kernel(table_hbm, batch_hbm, len_hbm, inv_hbm, out_hbm,
                idx_v, len_v, inv_v, pooled_v,
                rows0a, rows0b, rows1a, rows1b,
                sem0a, sem0b, sem1a, sem1b):
    wid = lax.axis_index("s") * NC + lax.axis_index("c")
    base = wid * NB

    pltpu.sync_copy(batch_hbm.at[pl.ds(base * L, NB * L)], idx_v)
    pltpu.sync_copy(len_hbm.at[pl.ds(base, NB)], len_v.at[pl.ds(0, NB)])
    pltpu.sync_copy(inv_hbm.at[pl.ds(base, NB)], inv_v.at[pl.ds(0, NB)])

    bufs = ((rows0a, rows0b, sem0a, sem0b), (rows1a, rows1b, sem1a, sem1b))

    def length_of(i):
        return len_v[pl.ds(i, LANES)][0]

    def issue(i, ra, rb, sa, sb):
        pltpu.async_copy(table_hbm.at[idx_v.at[pl.ds(i * L, C0)]], ra, sa)

        @pl.when(length_of(i) > C0)
        def _():
            pltpu.async_copy(table_hbm.at[idx_v.at[pl.ds(i * L + C0, C1)]],
                             rb, sb)

    def drain(i, ra, rb, sa, sb):
        pltpu.make_async_copy(
            table_hbm.at[idx_v.at[pl.ds(i * L, C0)]], ra, sa).wait()

        @pl.when(length_of(i) > C0)
        def _():
            pltpu.make_async_copy(
                table_hbm.at[idx_v.at[pl.ds(i * L + C0, C1)]], rb, sb).wait()

    issue(0, *bufs[0])

    def group_body(g, carry):
        for b in range(2):
            i = 2 * g + b
            ra, rb, sa, sb = bufs[b]
            nra, nrb, nsa, nsb = bufs[1 - b]

            @pl.when(i + 1 < NB)
            def _():
                issue(i + 1, nra, nrb, nsa, nsb)

            drain(i, ra, rb, sa, sb)

            length = length_of(i)
            n0 = jnp.minimum(length, C0)
            n1 = jnp.maximum(length - C0, 0)

            def acc0(j, acc, ra=ra):
                return tuple(acc[d] + ra[j, pl.ds(d * LANES, LANES)]
                             for d in range(ND))

            def acc1(j, acc, rb=rb):
                return tuple(acc[d] + rb[j, pl.ds(d * LANES, LANES)]
                             for d in range(ND))

            zeros = tuple(jnp.zeros((LANES,), jnp.float32)
                          for _ in range(ND))
            acc = lax.fori_loop(0, n0, acc0, zeros)
            acc = lax.fori_loop(0, n1, acc1, acc)

            inv = inv_v[pl.ds(i, LANES)][0]
            for d in range(ND):
                pooled_v[i, pl.ds(d * LANES, LANES)] = acc[d] * inv
        return carry

    lax.fori_loop(0, NB // 2, group_body, 0)
    pltpu.sync_copy(pooled_v, out_hbm.at[pl.ds(base, NB), :])


def _embedding_bag(table, batch, lengths, inv):
    mesh = plsc.VectorSubcoreMesh(core_axis_name="c", subcore_axis_name="s")
    f = functools.partial(
        pl.kernel,
        mesh=mesh,
        out_type=jax.ShapeDtypeStruct((B, D), jnp.float32),
        compiler_params=pltpu.CompilerParams(use_tc_tiling_on_sc=False),
        scratch_types=[
            pltpu.VMEM((NB * L,), jnp.int32),
            pltpu.VMEM((NB + LANES,), jnp.int32),
            pltpu.VMEM((NB + LANES,), jnp.float32),
            pltpu.VMEM((NB, D), jnp.float32),
            pltpu.VMEM((C0, D), jnp.float32),
            pltpu.VMEM((C1, D), jnp.float32),
            pltpu.VMEM((C0, D), jnp.float32),
            pltpu.VMEM((C1, D), jnp.float32),
            pltpu.SemaphoreType.DMA,
            pltpu.SemaphoreType.DMA,
            pltpu.SemaphoreType.DMA,
            pltpu.SemaphoreType.DMA,
        ],
    )(_bag_kernel)
    return f(table, batch, lengths, inv)


def _mlp_kernel(x_ref, w1_ref, b1_ref, w2_ref, b2_ref, o_ref):
    x = x_ref[...]
    h = lax.dot_general(x, w1_ref[...], (((1,), (1,)), ((), ())),
                        preferred_element_type=jnp.float32)
    h = jnp.maximum(h + b1_ref[...], 0.0)
    o = lax.dot_general(h, w2_ref[...], (((1,), (1,)), ((), ())),
                        preferred_element_type=jnp.float32)
    o_ref[...] = o + b2_ref[...]


def _mlp(pooled, W1, b1, W2, b2):
    return pl.pallas_call(
        _mlp_kernel,
        out_shape=jax.ShapeDtypeStruct((B, 2), jnp.float32),
    )(pooled, W1, b1.reshape(1, -1), W2, b2.reshape(1, -1))


@jax.jit
def kernel(table, batch, lengths, W1, b1, W2, b2):
    inv = 1.0 / jnp.clip(lengths, 1, None).astype(jnp.float32)
    pooled = _embedding_bag(table, batch.reshape(-1), lengths, inv)
    return _mlp(pooled, W1, b1, W2, b2)


# speedup vs baseline: 1.8995x; 1.4972x over previous
import functools

import jax
import jax.numpy as jnp
from jax import lax
from jax.experimental import pallas as pl
from jax.experimental.pallas import tpu as pltpu
from jax.experimental.pallas import tpu_sc as plsc

VOCAB = 1000000
D = 64
B = 4096
L = 200
PK = 2 * D
PBLK = 4096
H = 123 * PBLK

NC = 2
NS = 16
LANES = 16
NW = NC * NS
NB = B // NW
ND = D // LANES
CHUNKS = ((0, 56), (56, 48), (104, 48), (152, 48))
NCH = len(CHUNKS)


def _bag_kernel(table_hbm, pr_hbm, woff_hbm, len_hbm, inv_hbm, out_hbm,
                idx_v, woff_v, len_v, inv_v, pooled_v, *bufsem):
    rows = bufsem[:2 * NCH]
    sems = bufsem[2 * NCH:]
    bufs = tuple(
        tuple((rows[p * NCH + k], sems[p * NCH + k]) for k in range(NCH))
        for p in range(2))

    wid = lax.axis_index("s") * NC + lax.axis_index("c")
    base = wid * NB

    pltpu.sync_copy(pr_hbm.at[pl.ds(base * L, NB * L)],
                    idx_v.at[pl.ds(0, NB * L)])
    pltpu.sync_copy(woff_hbm.at[pl.ds(base * L, NB * L)],
                    woff_v.at[pl.ds(0, NB * L)])
    pltpu.sync_copy(len_hbm.at[pl.ds(base, NB)], len_v.at[pl.ds(0, NB)])
    pltpu.sync_copy(inv_hbm.at[pl.ds(base, NB)], inv_v.at[pl.ds(0, NB)])

    def length_of(i):
        return len_v[pl.ds(i, LANES)][0]

    def issue(i, pbufs):
        length = length_of(i)
        for k, (off, sz) in enumerate(CHUNKS):
            r, s = pbufs[k]

            def start(i=i, off=off, sz=sz, r=r, s=s):
                pltpu.async_copy(
                    table_hbm.at[idx_v.at[pl.ds(i * L + off, sz)]], r, s)

            if off == 0:
                start()
            else:
                pl.when(length > off)(start)

    def drain(i, pbufs):
        length = length_of(i)
        for k, (off, sz) in enumerate(CHUNKS):
            r, s = pbufs[k]

            def wait(i=i, off=off, sz=sz, r=r, s=s):
                pltpu.make_async_copy(
                    table_hbm.at[idx_v.at[pl.ds(i * L + off, sz)]],
                    r, s).wait()

            if off == 0:
                wait()
            else:
                pl.when(length > off)(wait)

    issue(0, bufs[0])

    def group_body(g, carry):
        for p in range(2):
            i = 2 * g + p
            pbufs = bufs[p]

            @pl.when(i + 1 < NB)
            def _():
                issue(i + 1, bufs[1 - p])

            drain(i, pbufs)

            length = length_of(i)
            acc = tuple(jnp.zeros((LANES,), jnp.float32)
                        for _ in range(ND))
            for k, (off, sz) in enumerate(CHUNKS):
                r = pbufs[k][0]
                nk = jnp.clip(length - off, 0, sz)

                def body(j, a, r=r, i=i, off=off):
                    w = woff_v[pl.ds(i * L + off + j, LANES)][0]
                    return tuple(a[d] + r[j, pl.ds(w + d * LANES, LANES)]
                                 for d in range(ND))

                acc = lax.fori_loop(0, nk, body, acc)

            inv = inv_v[pl.ds(i, LANES)][0]
            for d in range(ND):
                pooled_v[pl.ds(i * D + d * LANES, LANES)] = acc[d] * inv
        return carry

    lax.fori_loop(0, NB // 2, group_body, 0)
    pltpu.sync_copy(pooled_v, out_hbm.at[pl.ds(base * D, NB * D)])


def _embedding_bag(table_pk, pr_flat, woff_flat, lengths, inv):
    mesh = plsc.VectorSubcoreMesh(core_axis_name="c", subcore_axis_name="s")
    row_bufs = [pltpu.VMEM((sz, PK), jnp.float32)
                for _ in range(2) for (_, sz) in CHUNKS]
    dma_sems = [pltpu.SemaphoreType.DMA] * (2 * NCH)
    f = functools.partial(
        pl.kernel,
        mesh=mesh,
        out_type=jax.ShapeDtypeStruct((B * D,), jnp.float32),
        compiler_params=pltpu.CompilerParams(use_tc_tiling_on_sc=True),
        scratch_types=[
            pltpu.VMEM((NB * L,), jnp.int32),
            pltpu.VMEM((NB * L + LANES,), jnp.int32),
            pltpu.VMEM((NB + LANES,), jnp.int32),
            pltpu.VMEM((NB + LANES,), jnp.float32),
            pltpu.VMEM((NB * D,), jnp.float32),
        ] + row_bufs + dma_sems,
    )(_bag_kernel)
    return f(table_pk, pr_flat, woff_flat, lengths, inv)


def _pack_kernel(ta_ref, tb_ref, o_ref):
    o_ref[:, 0:D] = ta_ref[...].T
    o_ref[:, D:PK] = tb_ref[...].T


def _pack_table(tableT):
    return pl.pallas_call(
        _pack_kernel,
        grid=(H // PBLK,),
        in_specs=[
            pl.BlockSpec((D, PBLK), lambda i: (0, i)),
            pl.BlockSpec((D, PBLK),
                         lambda i: (0, jnp.minimum(i + H // PBLK,
                                                   VOCAB // PBLK))),
        ],
        out_specs=pl.BlockSpec((PBLK, PK), lambda i: (i, 0)),
        out_shape=jax.ShapeDtypeStruct((H, PK), jnp.float32),
    )(tableT, tableT)


def _mlp_kernel(x_ref, w1_ref, b1_ref, w2_ref, b2_ref, o_ref):
    x = x_ref[...]
    h = lax.dot_general(x, w1_ref[...], (((1,), (1,)), ((), ())),
                        preferred_element_type=jnp.float32)
    h = jnp.maximum(h + b1_ref[...], 0.0)
    o = lax.dot_general(h, w2_ref[...], (((1,), (1,)), ((), ())),
                        preferred_element_type=jnp.float32)
    o_ref[...] = o + b2_ref[...]


def _mlp(pooled, W1, b1, W2, b2):
    return pl.pallas_call(
        _mlp_kernel,
        out_shape=jax.ShapeDtypeStruct((B, 2), jnp.float32),
    )(pooled, W1, b1.reshape(1, -1), W2, b2.reshape(1, -1))


@jax.jit
def kernel(table, batch, lengths, W1, b1, W2, b2):
    inv = 1.0 / jnp.clip(lengths, 1, None).astype(jnp.float32)
    table_pk = _pack_table(table.T)
    hi = batch >= H
    pr_flat = jnp.where(hi, batch - H, batch).reshape(-1)
    woff_flat = (hi.astype(jnp.int32) << 6).reshape(-1)
    pooled = _embedding_bag(table_pk, pr_flat, woff_flat, lengths, inv)
    return _mlp(pooled.reshape(B, D), W1, b1, W2, b2)


# speedup vs baseline: 2.0947x; 1.1028x over previous
import functools

import jax
import jax.numpy as jnp
from jax import lax
from jax.experimental import pallas as pl
from jax.experimental.pallas import tpu as pltpu
from jax.experimental.pallas import tpu_sc as plsc

VOCAB = 1000000
D = 64
B = 4096
L = 200
PK = 2 * D
PBLK = 8192
H = 62 * PBLK

NC = 2
NS = 16
LANES = 16
NW = NC * NS
NB = B // NW
ND = D // LANES
CHUNKS = ((0, 56), (56, 48), (104, 48), (152, 48))
NCH = len(CHUNKS)


def _bag_kernel(table_hbm, batch_hbm, len_hbm, inv_hbm, out_hbm,
                idx_v, woff_v, len_v, inv_v, pooled_v, *bufsem):
    rows = bufsem[:2 * NCH]
    sems = bufsem[2 * NCH:]
    bufs = tuple(
        tuple((rows[p * NCH + k], sems[p * NCH + k]) for k in range(NCH))
        for p in range(2))

    wid = lax.axis_index("s") * NC + lax.axis_index("c")
    base = wid * NB

    pltpu.sync_copy(batch_hbm.at[pl.ds(base * L, NB * L)],
                    idx_v.at[pl.ds(0, NB * L)])
    pltpu.sync_copy(len_hbm.at[pl.ds(base, NB)], len_v.at[pl.ds(0, NB)])
    pltpu.sync_copy(inv_hbm.at[pl.ds(base, NB)], inv_v.at[pl.ds(0, NB)])

    def transform(t, carry):
        v = idx_v[pl.ds(t * LANES, LANES)]
        hi = v >= H
        idx_v[pl.ds(t * LANES, LANES)] = jnp.where(hi, v - H, v)
        woff_v[pl.ds(t * LANES, LANES)] = jnp.where(hi, D, 0)
        return carry

    lax.fori_loop(0, L // LANES + 1, transform, 0)

    def length_of(i):
        return len_v[pl.ds(i, LANES)][0]

    def issue(i, pbufs):
        length = length_of(i)
        for k, (off, sz) in enumerate(CHUNKS):
            r, s = pbufs[k]

            def start(i=i, off=off, sz=sz, r=r, s=s):
                pltpu.async_copy(
                    table_hbm.at[idx_v.at[pl.ds(i * L + off, sz)]], r, s)

            if off == 0:
                start()
            else:
                pl.when(length > off)(start)

    def drain(i, pbufs):
        length = length_of(i)
        for k, (off, sz) in enumerate(CHUNKS):
            r, s = pbufs[k]

            def wait(i=i, off=off, sz=sz, r=r, s=s):
                pltpu.make_async_copy(
                    table_hbm.at[idx_v.at[pl.ds(i * L + off, sz)]],
                    r, s).wait()

            if off == 0:
                wait()
            else:
                pl.when(length > off)(wait)

    issue(0, bufs[0])
    lax.fori_loop(L // LANES + 1, NB * L // LANES, transform, 0)

    def group_body(g, carry):
        for p in range(2):
            i = 2 * g + p
            pbufs = bufs[p]

            @pl.when(i + 1 < NB)
            def _():
                issue(i + 1, bufs[1 - p])

            drain(i, pbufs)

            length = length_of(i)
            acc = tuple(jnp.zeros((LANES,), jnp.float32)
                        for _ in range(ND))
            for k, (off, sz) in enumerate(CHUNKS):
                r = pbufs[k][0]
                nk = jnp.clip(length - off, 0, sz)

                def body(j, a, r=r, i=i, off=off):
                    w = woff_v[pl.ds(i * L + off + j, LANES)][0]
                    return tuple(a[d] + r[j, pl.ds(w + d * LANES, LANES)]
                                 for d in range(ND))

                acc = lax.fori_loop(0, nk, body, acc)

            inv = inv_v[pl.ds(i, LANES)][0]
            for d in range(ND):
                pooled_v[pl.ds(i * D + d * LANES, LANES)] = acc[d] * inv
        return carry

    lax.fori_loop(0, NB // 2, group_body, 0)
    pltpu.sync_copy(pooled_v, out_hbm.at[pl.ds(base * D, NB * D)])


def _embedding_bag(table_pk, batch_flat, lengths, inv):
    mesh = plsc.VectorSubcoreMesh(core_axis_name="c", subcore_axis_name="s")
    row_bufs = [pltpu.VMEM((sz, PK), jnp.float32)
                for _ in range(2) for (_, sz) in CHUNKS]
    dma_sems = [pltpu.SemaphoreType.DMA] * (2 * NCH)
    f = functools.partial(
        pl.kernel,
        mesh=mesh,
        out_type=jax.ShapeDtypeStruct((B * D,), jnp.float32),
        compiler_params=pltpu.CompilerParams(use_tc_tiling_on_sc=True),
        scratch_types=[
            pltpu.VMEM((NB * L,), jnp.int32),
            pltpu.VMEM((NB * L + LANES,), jnp.int32),
            pltpu.VMEM((NB + LANES,), jnp.int32),
            pltpu.VMEM((NB + LANES,), jnp.float32),
            pltpu.VMEM((NB * D,), jnp.float32),
        ] + row_bufs + dma_sems,
    )(_bag_kernel)
    return f(table_pk, batch_flat, lengths, inv)


def _pack_kernel(ta_ref, tb_ref, o_ref):
    eye = (lax.broadcasted_iota(jnp.int32, (D, D), 0)
           == lax.broadcasted_iota(jnp.int32, (D, D), 1)).astype(jnp.float32)
    ta = lax.dot_general(ta_ref[...], eye, (((0,), (0,)), ((), ())),
                         preferred_element_type=jnp.float32)
    tb = lax.dot_general(tb_ref[...], eye, (((0,), (0,)), ((), ())),
                         preferred_element_type=jnp.float32)
    o_ref[...] = jnp.concatenate([ta, tb], axis=1)


def _pack_table(tableT):
    return pl.pallas_call(
        _pack_kernel,
        grid=(H // PBLK,),
        in_specs=[
            pl.BlockSpec((D, PBLK), lambda i: (0, i)),
            pl.BlockSpec((D, PBLK),
                         lambda i: (0, jnp.minimum(i + H // PBLK,
                                                   VOCAB // PBLK))),
        ],
        out_specs=pl.BlockSpec((PBLK, PK), lambda i: (i, 0)),
        out_shape=jax.ShapeDtypeStruct((H, PK), jnp.float32),
        compiler_params=pltpu.CompilerParams(
            dimension_semantics=("parallel",)),
    )(tableT, tableT)


def _mlp_kernel(x_ref, w1_ref, b1_ref, w2_ref, b2_ref, o_ref):
    x = x_ref[...]
    h = lax.dot_general(x, w1_ref[...], (((1,), (1,)), ((), ())),
                        preferred_element_type=jnp.float32)
    h = jnp.maximum(h + b1_ref[...], 0.0)
    o = lax.dot_general(h, w2_ref[...], (((1,), (1,)), ((), ())),
                        preferred_element_type=jnp.float32)
    o_ref[...] = o + b2_ref[...]


def _mlp(pooled, W1, b1, W2, b2):
    return pl.pallas_call(
        _mlp_kernel,
        out_shape=jax.ShapeDtypeStruct((B, 2), jnp.float32),
    )(pooled, W1, b1.reshape(1, -1), W2, b2.reshape(1, -1))


@jax.jit
def kernel(table, batch, lengths, W1, b1, W2, b2):
    inv = 1.0 / jnp.clip(lengths, 1, None).astype(jnp.float32)
    table_pk = _pack_table(table.T)
    pooled = _embedding_bag(table_pk, batch.reshape(-1), lengths, inv)
    return _mlp(pooled.reshape(B, D), W1, b1, W2, b2)
